# baseline (speedup 1.0000x reference)
import functools

import jax
import jax.numpy as jnp
from jax import lax
from jax.experimental import pallas as pl
from jax.experimental.pallas import tpu as pltpu
from jax.experimental.pallas import tpu_sc as plsc

N_ATOMS = 10000
N_BONDS = 160000
D = 16
DP = 128
NW = 32
P = 320
NPAD = NW * P
C = 128
G = C // 16

_mesh = plsc.VectorSubcoreMesh(
    core_axis_name="c", subcore_axis_name="s", num_cores=2, num_subcores=16
)

_GATHER_DNUMS = lax.GatherDimensionNumbers(
    offset_dims=(), collapsed_slice_dims=(0,), start_index_map=(0,)
)


def _dyn_gather(v, idx):
    return lax.gather(
        v, idx[:, None], _GATHER_DNUMS, (1,),
        mode=lax.GatherScatterMode.PROMISE_IN_BOUNDS,
    )


@functools.partial(
    pl.kernel,
    out_type=jax.ShapeDtypeStruct((NPAD * D,), jnp.float32),
    mesh=_mesh,
    compiler_params=pltpu.CompilerParams(
        needs_layout_passes=False, use_tc_tiling_on_sc=True
    ),
    scratch_types=[
        pltpu.VMEM((16,), jnp.int32),
        [pltpu.VMEM((C,), jnp.int32)] * 2,
        [pltpu.VMEM((C,), jnp.int32)] * 2,
        [pltpu.VMEM((C, DP), jnp.float32)] * 2,
        [pltpu.VMEM((D, D, C), jnp.float32)] * 2,
        pltpu.VMEM((P * D,), jnp.float32),
        pltpu.VMEM((G * 17 * D,), jnp.float32),
        pltpu.VMEM((G * 17 * D,), jnp.float32),
        [pltpu.SemaphoreType.DMA] * 2,
        [pltpu.SemaphoreType.DMA] * 2,
    ],
)
def _sc_message_sum(atom_hbm, bondT_hbm, src_hbm, dst_hbm, bounds_hbm,
                    out_hbm, bounds_v, src_v, dst_v, atoms_v, bond_v,
                    win_v, at_t, msg_t, sem_a, sem_b):
    wid = lax.axis_index("c") * 16 + lax.axis_index("s")
    lane = lax.iota(jnp.int32, 16)

    pltpu.sync_copy(bounds_hbm.at[pl.ds(wid * 16, 16)], bounds_v)
    bv = bounds_v[...]
    e_lo = jnp.sum(jnp.where(lane == 0, bv, 0))
    e_hi = jnp.sum(jnp.where(lane == 1, bv, 0))
    base_atom = wid * P

    @plsc.parallel_loop(0, P, unroll=4)
    def zero_body(j):
        win_v[pl.ds(j * 16, 16)] = jnp.zeros((16,), jnp.float32)

    e128 = jnp.bitwise_and(e_lo, -128)
    n_chunks = jnp.right_shift(e_hi - e128 + (C - 1), 7)

    def chunk_base(ci):
        return pl.multiple_of(jnp.minimum(e128 + ci * C, N_BONDS - C), C)

    def fire(ci, s):
        b = chunk_base(ci)
        pltpu.sync_copy(src_hbm.at[pl.ds(b, C)], src_v[s])
        pltpu.sync_copy(dst_hbm.at[pl.ds(b, C)], dst_v[s])
        pltpu.async_copy(atom_hbm.at[src_v[s]], atoms_v[s], sem_a[s])
        pltpu.async_copy(bondT_hbm.at[:, :, pl.ds(b, C)], bond_v[s], sem_b[s])

    def drain(ci, s):
        b = chunk_base(ci)
        pltpu.make_async_copy(atom_hbm.at[src_v[s]], atoms_v[s],
                              sem_a[s]).wait()
        pltpu.make_async_copy(bondT_hbm.at[:, :, pl.ds(b, C)], bond_v[s],
                              sem_b[s]).wait()

    def compute(ci, s):
        start = e128 + ci * C
        resp_lo = jnp.maximum(e_lo, start)
        resp_hi = jnp.minimum(e_hi, start + C)
        base = chunk_base(ci)
        dst_s, atoms_s, bond_s = dst_v[s], atoms_v[s], bond_v[s]

        @plsc.parallel_loop(0, G, unroll=4)
        def group_body(g):
            eb = g * 16
            dst_g = dst_s[pl.ds(eb, 16)]
            rel16 = (dst_g - base_atom) * 16
            ev = base + eb + lane
            vmask = (ev >= resp_lo) & (ev < resp_hi)
            tb = g * (17 * D)
            for k in range(16):
                a_vec = atoms_s[eb + k, pl.ds(0, 16)]
                plsc.store_scatter(at_t, [lane * 17 + (tb + k)], a_vec)
            atjs = [at_t[pl.ds(tb + j * 17, 16)] for j in range(D)]
            vmi = jnp.where(vmask, 1, 0)
            for i in range(D):
                acc = bond_s[i, 0, pl.ds(eb, 16)] * atjs[0]
                for j in range(1, D):
                    acc = acc + bond_s[i, j, pl.ds(eb, 16)] * atjs[j]
                plsc.store_scatter(msg_t, [lane * 17 + (tb + i)], acc)
            for k in range(16):
                m_k = msg_t[pl.ds(tb + k * 17, 16)]
                ksp = jnp.full((16,), k, jnp.int32)
                dsp = _dyn_gather(rel16, ksp)
                okk = (_dyn_gather(vmi, ksp) > 0)
                plsc.addupdate_scatter(win_v, [dsp + lane], m_k, mask=okk)

    @pl.when(n_chunks > 0)
    def _():
        fire(0, 0)

    def pair_body(pi, _):
        for s in (0, 1):
            ci = 2 * pi + s

            @pl.when(ci < n_chunks)
            def _(ci=ci, s=s):
                @pl.when(ci + 1 < n_chunks)
                def _():
                    fire(ci + 1, 1 - s)
                drain(ci, s)
                compute(ci, s)
        return 0

    lax.fori_loop(0, jnp.right_shift(n_chunks + 1, 1), pair_body, 0)

    pltpu.sync_copy(win_v, out_hbm.at[pl.ds(wid * (P * D), P * D)])


def kernel(atom_matrix, bond_matrix, connectivity):
    src = connectivity[:, 1].astype(jnp.int32)
    dst = connectivity[:, 0].astype(jnp.int32)
    bond_t = jnp.transpose(bond_matrix, (1, 2, 0))
    atom_pad = jnp.pad(atom_matrix, ((0, 0), (0, DP - D)))
    cuts = jnp.arange(NW + 1, dtype=jnp.int32) * P
    edges = jnp.searchsorted(dst, cuts, side="left").astype(jnp.int32)
    bounds = jnp.zeros((NW, 16), jnp.int32)
    bounds = bounds.at[:, 0].set(edges[:-1]).at[:, 1].set(edges[1:])
    out = _sc_message_sum(atom_pad, bond_t, src, dst,
                          bounds.reshape(-1))
    return out.reshape(NPAD, D)[:N_ATOMS]


# speedup vs baseline: 2.1846x; 2.1846x over previous
import functools

import jax
import jax.numpy as jnp
from jax import lax
from jax.experimental import pallas as pl
from jax.experimental.pallas import tpu as pltpu
from jax.experimental.pallas import tpu_sc as plsc

N_ATOMS = 10000
N_BONDS = 160000
D = 16
DP = 128
NW = 32
P = 320
NPAD = NW * P
C = 128
G = C // 16

_mesh = plsc.VectorSubcoreMesh(
    core_axis_name="c", subcore_axis_name="s", num_cores=2, num_subcores=16
)

_GATHER_DNUMS = lax.GatherDimensionNumbers(
    offset_dims=(), collapsed_slice_dims=(0,), start_index_map=(0,)
)


def _dyn_gather(v, idx):
    return lax.gather(
        v, idx[:, None], _GATHER_DNUMS, (1,),
        mode=lax.GatherScatterMode.PROMISE_IN_BOUNDS,
    )


@functools.partial(
    pl.kernel,
    out_type=jax.ShapeDtypeStruct((NPAD * D,), jnp.float32),
    mesh=_mesh,
    compiler_params=pltpu.CompilerParams(
        needs_layout_passes=False, use_tc_tiling_on_sc=True
    ),
    scratch_types=[
        pltpu.VMEM((16,), jnp.int32),
        [pltpu.VMEM((C,), jnp.int32)] * 4,
        [pltpu.VMEM((C,), jnp.int32)] * 4,
        [pltpu.VMEM((C, DP), jnp.float32)] * 2,
        [pltpu.VMEM((D, D, C), jnp.float32)] * 2,
        pltpu.VMEM((P * D,), jnp.float32),
        pltpu.VMEM((G * 17 * D,), jnp.float32),
        pltpu.VMEM((G * 17 * D,), jnp.float32),
        [pltpu.SemaphoreType.DMA] * 2,
        [pltpu.SemaphoreType.DMA] * 2,
        [pltpu.SemaphoreType.DMA] * 4,
    ],
)
def _sc_message_sum(atom_hbm, bondT_hbm, src_hbm, dst_hbm, bounds_hbm,
                    out_hbm, bounds_v, src_v, dst_v, atoms_v, bond_v,
                    win_v, at_t, msg_t, sem_a, sem_b, sem_sd):
    wid = lax.axis_index("c") * 16 + lax.axis_index("s")
    lane = lax.iota(jnp.int32, 16)

    pltpu.sync_copy(bounds_hbm.at[pl.ds(wid * 16, 16)], bounds_v)
    bv = bounds_v[...]
    e_lo = jnp.sum(jnp.where(lane == 0, bv, 0))
    e_hi = jnp.sum(jnp.where(lane == 1, bv, 0))
    base_atom = wid * P

    @plsc.parallel_loop(0, P, unroll=4)
    def zero_body(j):
        win_v[pl.ds(j * 16, 16)] = jnp.zeros((16,), jnp.float32)

    e128 = jnp.bitwise_and(e_lo, -128)
    n_chunks = jnp.right_shift(e_hi - e128 + (C - 1), 7)

    def chunk_base(ci):
        return pl.multiple_of(jnp.minimum(e128 + ci * C, N_BONDS - C), C)

    def fire_sd(ci, q):
        b = chunk_base(ci)
        pltpu.async_copy(src_hbm.at[pl.ds(b, C)], src_v[q], sem_sd[q])
        pltpu.async_copy(dst_hbm.at[pl.ds(b, C)], dst_v[q], sem_sd[q])

    def drain_sd(ci, q):
        b = chunk_base(ci)
        pltpu.make_async_copy(src_hbm.at[pl.ds(b, C)], src_v[q],
                              sem_sd[q]).wait()
        pltpu.make_async_copy(dst_hbm.at[pl.ds(b, C)], dst_v[q],
                              sem_sd[q]).wait()

    def fire_ab(ci, s, q):
        b = chunk_base(ci)
        pltpu.async_copy(atom_hbm.at[src_v[q]], atoms_v[s], sem_a[s])
        pltpu.async_copy(bondT_hbm.at[:, :, pl.ds(b, C)], bond_v[s], sem_b[s])

    def drain_ab(ci, s, q):
        b = chunk_base(ci)
        pltpu.make_async_copy(atom_hbm.at[src_v[q]], atoms_v[s],
                              sem_a[s]).wait()
        pltpu.make_async_copy(bondT_hbm.at[:, :, pl.ds(b, C)], bond_v[s],
                              sem_b[s]).wait()

    def compute(ci, s):
        start = e128 + ci * C
        resp_lo = jnp.maximum(e_lo, start)
        resp_hi = jnp.minimum(e_hi, start + C)
        base = chunk_base(ci)
        dst_s, atoms_s, bond_s = dst_v[s & 3], atoms_v[s & 1], bond_v[s & 1]

        @plsc.parallel_loop(0, G, unroll=2)
        def group_body(g):
            eb = g * 16
            dst_g = dst_s[pl.ds(eb, 16)]
            rel16 = (dst_g - base_atom) * 16
            ev = base + eb + lane
            vmask = (ev >= resp_lo) & (ev < resp_hi)
            tb = g * (17 * D)
            for k in range(16):
                a_vec = atoms_s[eb + k, pl.ds(0, 16)]
                plsc.store_scatter(at_t, [lane * 17 + (tb + k)], a_vec)
            atjs = [at_t[pl.ds(tb + j * 17, 16)] for j in range(D)]
            vmi = jnp.where(vmask, 1, 0)
            for i in range(D):
                acc = bond_s[i, 0, pl.ds(eb, 16)] * atjs[0]
                for j in range(1, D):
                    acc = acc + bond_s[i, j, pl.ds(eb, 16)] * atjs[j]
                plsc.store_scatter(msg_t, [lane * 17 + (tb + i)], acc)
            for k in range(16):
                m_k = msg_t[pl.ds(tb + k * 17, 16)]
                ksp = jnp.full((16,), k, jnp.int32)
                dsp = _dyn_gather(rel16, ksp)
                okk = (_dyn_gather(vmi, ksp) > 0)
                plsc.addupdate_scatter(win_v, [dsp + lane], m_k, mask=okk)

    @pl.when(n_chunks > 0)
    def _():
        b0 = chunk_base(0)
        pltpu.sync_copy(src_hbm.at[pl.ds(b0, C)], src_v[0])
        pltpu.sync_copy(dst_hbm.at[pl.ds(b0, C)], dst_v[0])
        fire_ab(0, 0, 0)

        @pl.when(n_chunks > 1)
        def _():
            fire_sd(1, 1)

    def quad_body(qi, _):
        for s in range(4):
            ci = 4 * qi + s

            @pl.when(ci < n_chunks)
            def _(ci=ci, s=s):
                @pl.when(ci + 1 < n_chunks)
                def _():
                    drain_sd(ci + 1, (s + 1) & 3)
                    fire_ab(ci + 1, (s + 1) & 1, (s + 1) & 3)

                @pl.when(ci + 2 < n_chunks)
                def _():
                    fire_sd(ci + 2, (s + 2) & 3)

                drain_ab(ci, s & 1, s & 3)
                compute(ci, s)
        return 0

    lax.fori_loop(0, jnp.right_shift(n_chunks + 3, 2), quad_body, 0)

    pltpu.sync_copy(win_v, out_hbm.at[pl.ds(wid * (P * D), P * D)])


def kernel(atom_matrix, bond_matrix, connectivity):
    src = connectivity[:, 1].astype(jnp.int32)
    dst = connectivity[:, 0].astype(jnp.int32)
    bond_t = jnp.transpose(bond_matrix, (1, 2, 0))
    atom_pad = jnp.pad(atom_matrix, ((0, 0), (0, DP - D)))
    cuts = jnp.arange(NW + 1, dtype=jnp.int32) * P
    edges = jnp.searchsorted(dst, cuts, side="left").astype(jnp.int32)
    bounds = jnp.zeros((NW, 16), jnp.int32)
    bounds = bounds.at[:, 0].set(edges[:-1]).at[:, 1].set(edges[1:])
    out = _sc_message_sum(atom_pad, bond_t, src, dst,
                          bounds.reshape(-1))
    return out.reshape(NPAD, D)[:N_ATOMS]


# speedup vs baseline: 2.2376x; 1.0242x over previous
import functools

import jax
import jax.numpy as jnp
from jax import lax
from jax.experimental import pallas as pl
from jax.experimental.pallas import tpu as pltpu
from jax.experimental.pallas import tpu_sc as plsc

N_ATOMS = 10000
N_BONDS = 160000
D = 16
DP = 128
NW = 32
P = 320
NPAD = NW * P
C = 128
G = C // 16

_mesh = plsc.VectorSubcoreMesh(
    core_axis_name="c", subcore_axis_name="s", num_cores=2, num_subcores=16
)

_GATHER_DNUMS = lax.GatherDimensionNumbers(
    offset_dims=(), collapsed_slice_dims=(0,), start_index_map=(0,)
)


def _dyn_gather(v, idx):
    return lax.gather(
        v, idx[:, None], _GATHER_DNUMS, (1,),
        mode=lax.GatherScatterMode.PROMISE_IN_BOUNDS,
    )


@functools.partial(
    pl.kernel,
    out_type=jax.ShapeDtypeStruct((NPAD * D,), jnp.float32),
    mesh=_mesh,
    compiler_params=pltpu.CompilerParams(
        needs_layout_passes=False, use_tc_tiling_on_sc=True
    ),
    scratch_types=[
        pltpu.VMEM((16,), jnp.int32),
        [pltpu.VMEM((C,), jnp.int32)] * 4,
        [pltpu.VMEM((C,), jnp.int32)] * 4,
        [pltpu.VMEM((C, D), jnp.float32)] * 2,
        pltpu.VMEM_SHARED((N_ATOMS, D), jnp.float32),
        [pltpu.VMEM((D, D, C), jnp.float32)] * 2,
        pltpu.VMEM((P * D,), jnp.float32),
        pltpu.VMEM((G * 17 * D,), jnp.float32),
        pltpu.VMEM((G * 17 * D,), jnp.float32),
        [pltpu.SemaphoreType.DMA] * 2,
        [pltpu.SemaphoreType.DMA] * 2,
        [pltpu.SemaphoreType.DMA] * 4,
    ],
)
def _sc_message_sum(atom_hbm, bondT_hbm, src_hbm, dst_hbm, bounds_hbm,
                    out_hbm, bounds_v, src_v, dst_v, atoms_v, shared_atoms,
                    bond_v, win_v, at_t, msg_t, sem_a, sem_b, sem_sd):
    sid = lax.axis_index("s")
    wid = lax.axis_index("c") * 16 + sid
    lane = lax.iota(jnp.int32, 16)

    @pl.when(sid == 0)
    def _():
        pltpu.sync_copy(atom_hbm, shared_atoms)

    plsc.subcore_barrier()

    pltpu.sync_copy(bounds_hbm.at[pl.ds(wid * 16, 16)], bounds_v)
    bv = bounds_v[...]
    e_lo = jnp.sum(jnp.where(lane == 0, bv, 0))
    e_hi = jnp.sum(jnp.where(lane == 1, bv, 0))
    base_atom = wid * P

    @plsc.parallel_loop(0, P, unroll=4)
    def zero_body(j):
        win_v[pl.ds(j * 16, 16)] = jnp.zeros((16,), jnp.float32)

    e128 = jnp.bitwise_and(e_lo, -128)
    n_chunks = jnp.right_shift(e_hi - e128 + (C - 1), 7)

    def chunk_base(ci):
        return pl.multiple_of(jnp.minimum(e128 + ci * C, N_BONDS - C), C)

    def fire_sd(ci, q):
        b = chunk_base(ci)
        pltpu.async_copy(src_hbm.at[pl.ds(b, C)], src_v[q], sem_sd[q])
        pltpu.async_copy(dst_hbm.at[pl.ds(b, C)], dst_v[q], sem_sd[q])

    def drain_sd(ci, q):
        b = chunk_base(ci)
        pltpu.make_async_copy(src_hbm.at[pl.ds(b, C)], src_v[q],
                              sem_sd[q]).wait()
        pltpu.make_async_copy(dst_hbm.at[pl.ds(b, C)], dst_v[q],
                              sem_sd[q]).wait()

    def fire_ab(ci, s, q):
        b = chunk_base(ci)
        pltpu.async_copy(shared_atoms.at[src_v[q]], atoms_v[s], sem_a[s])
        pltpu.async_copy(bondT_hbm.at[:, :, pl.ds(b, C)], bond_v[s], sem_b[s])

    def drain_ab(ci, s, q):
        b = chunk_base(ci)
        pltpu.make_async_copy(shared_atoms.at[src_v[q]], atoms_v[s],
                              sem_a[s]).wait()
        pltpu.make_async_copy(bondT_hbm.at[:, :, pl.ds(b, C)], bond_v[s],
                              sem_b[s]).wait()

    def compute(ci, s):
        start = e128 + ci * C
        resp_lo = jnp.maximum(e_lo, start)
        resp_hi = jnp.minimum(e_hi, start + C)
        base = chunk_base(ci)
        dst_s, atoms_s, bond_s = dst_v[s & 3], atoms_v[s & 1], bond_v[s & 1]

        @plsc.parallel_loop(0, G, unroll=2)
        def group_body(g):
            eb = g * 16
            dst_g = dst_s[pl.ds(eb, 16)]
            rel16 = (dst_g - base_atom) * 16
            ev = base + eb + lane
            vmask = (ev >= resp_lo) & (ev < resp_hi)
            tb = g * (17 * D)
            for k in range(16):
                a_vec = atoms_s[eb + k, pl.ds(0, 16)]
                plsc.store_scatter(at_t, [lane * 17 + (tb + k)], a_vec)
            atjs = [at_t[pl.ds(tb + j * 17, 16)] for j in range(D)]
            vmi = jnp.where(vmask, 1, 0)
            for i in range(D):
                acc = bond_s[i, 0, pl.ds(eb, 16)] * atjs[0]
                for j in range(1, D):
                    acc = acc + bond_s[i, j, pl.ds(eb, 16)] * atjs[j]
                plsc.store_scatter(msg_t, [lane * 17 + (tb + i)], acc)
            for k in range(16):
                m_k = msg_t[pl.ds(tb + k * 17, 16)]
                ksp = jnp.full((16,), k, jnp.int32)
                dsp = _dyn_gather(rel16, ksp)
                okk = (_dyn_gather(vmi, ksp) > 0)
                plsc.addupdate_scatter(win_v, [dsp + lane], m_k, mask=okk)

    @pl.when(n_chunks > 0)
    def _():
        b0 = chunk_base(0)
        pltpu.sync_copy(src_hbm.at[pl.ds(b0, C)], src_v[0])
        pltpu.sync_copy(dst_hbm.at[pl.ds(b0, C)], dst_v[0])
        fire_ab(0, 0, 0)

        @pl.when(n_chunks > 1)
        def _():
            fire_sd(1, 1)

    def quad_body(qi, _):
        for s in range(4):
            ci = 4 * qi + s

            @pl.when(ci < n_chunks)
            def _(ci=ci, s=s):
                @pl.when(ci + 1 < n_chunks)
                def _():
                    drain_sd(ci + 1, (s + 1) & 3)
                    fire_ab(ci + 1, (s + 1) & 1, (s + 1) & 3)

                @pl.when(ci + 2 < n_chunks)
                def _():
                    fire_sd(ci + 2, (s + 2) & 3)

                drain_ab(ci, s & 1, s & 3)
                compute(ci, s)
        return 0

    lax.fori_loop(0, jnp.right_shift(n_chunks + 3, 2), quad_body, 0)

    pltpu.sync_copy(win_v, out_hbm.at[pl.ds(wid * (P * D), P * D)])


def kernel(atom_matrix, bond_matrix, connectivity):
    src = connectivity[:, 1].astype(jnp.int32)
    dst = connectivity[:, 0].astype(jnp.int32)
    bond_t = jnp.transpose(bond_matrix, (1, 2, 0))
    cuts = jnp.arange(NW + 1, dtype=jnp.int32) * P
    edges = jnp.searchsorted(dst, cuts, side="left").astype(jnp.int32)
    bounds = jnp.zeros((NW, 16), jnp.int32)
    bounds = bounds.at[:, 0].set(edges[:-1]).at[:, 1].set(edges[1:])
    out = _sc_message_sum(atom_matrix, bond_t, src, dst,
                          bounds.reshape(-1))
    return out.reshape(NPAD, D)[:N_ATOMS]
